# baseline (speedup 1.0000x reference)
import functools

import jax
import jax.numpy as jnp
from jax.experimental import pallas as pl
from jax.experimental.pallas import tpu as pltpu


def _prologue_body(x_ref, wmlp_ref, wA_ref, wAs_ref,
                   mlp_ref, xwA_ref, xwAs_ref):
    x = x_ref[...]
    mlp_ref[...] = jnp.maximum(
        jnp.dot(x, wmlp_ref[...], preferred_element_type=jnp.float32), 0.0)
    xwA_ref[...] = jnp.dot(
        x, wA_ref[...], preferred_element_type=jnp.float32).astype(jnp.bfloat16)
    xwAs_ref[...] = jnp.dot(
        x, wAs_ref[...], preferred_element_type=jnp.float32).astype(jnp.bfloat16)


def _main_body(adj_ref, sadj_ref, xwA_ref, xwAs_ref, mlp_ref,
               outA_ref, outAs_ref, colsum_ref):
    a = jnp.maximum(
        jnp.dot(adj_ref[...].astype(jnp.bfloat16), xwA_ref[...],
                preferred_element_type=jnp.float32), 0.0)
    b = jnp.maximum(
        jnp.dot(sadj_ref[...].astype(jnp.bfloat16), xwAs_ref[...],
                preferred_element_type=jnp.float32), 0.0)
    outA_ref[...] = a
    outAs_ref[...] = b
    part = jnp.sum(a + b + mlp_ref[...], axis=0, keepdims=True)
    i = pl.program_id(0)

    @pl.when(i == 0)
    def _set():
        colsum_ref[0:1, :] = part

    @pl.when(i > 0)
    def _add():
        colsum_ref[0:1, :] += part


def _attn_body(n_total, mlp_ref, outA_ref, outAs_ref, colsum_ref,
               attk_ref, attv_ref, emb_ref):
    tao = 3.0
    kvec = jnp.dot(colsum_ref[0:1, :] * (1.0 / n_total), attk_ref[...],
                   preferred_element_type=jnp.float32)
    mlp = mlp_ref[...]
    oA = outA_ref[...]
    oAs = outAs_ref[...]
    s0 = jnp.sum(mlp * kvec, axis=1, keepdims=True)
    s1 = jnp.sum(oA * kvec, axis=1, keepdims=True)
    s2 = jnp.sum(oAs * kvec, axis=1, keepdims=True)
    g0 = jax.nn.sigmoid(s0)
    g1 = jax.nn.sigmoid(s1)
    g2 = jax.nn.sigmoid(s2)
    v = attv_ref
    t0 = (g0 * v[0:1, 0:1] + g1 * v[1:2, 0:1] + g2 * v[2:3, 0:1]) * (1.0 / tao)
    t1 = (g0 * v[0:1, 1:2] + g1 * v[1:2, 1:2] + g2 * v[2:3, 1:2]) * (1.0 / tao)
    t2 = (g0 * v[0:1, 2:3] + g1 * v[1:2, 2:3] + g2 * v[2:3, 2:3]) * (1.0 / tao)
    m = jnp.maximum(t0, jnp.maximum(t1, t2))
    e0 = jnp.exp(t0 - m)
    e1 = jnp.exp(t1 - m)
    e2 = jnp.exp(t2 - m)
    den = e0 + e1 + e2
    emb_ref[...] = (e0 * mlp + e1 * oA + e2 * oAs) / den


def kernel(inputx, adj, sadj, weight_mlp, weight_A, weight_As,
           att_vec_k, att_vec_v):
    n, d = inputx.shape

    bm_pro = n // 5 if n % 5 == 0 else n
    bm = 200 if n % 200 == 0 else n
    ni = n // bm

    mlp, xwA, xwAs = pl.pallas_call(
        _prologue_body,
        grid=(n // bm_pro,),
        in_specs=[
            pl.BlockSpec((bm_pro, d), lambda i: (i, 0)),
            pl.BlockSpec((d, d), lambda i: (0, 0)),
            pl.BlockSpec((d, d), lambda i: (0, 0)),
            pl.BlockSpec((d, d), lambda i: (0, 0)),
        ],
        out_specs=[
            pl.BlockSpec((bm_pro, d), lambda i: (i, 0)),
            pl.BlockSpec((bm_pro, d), lambda i: (i, 0)),
            pl.BlockSpec((bm_pro, d), lambda i: (i, 0)),
        ],
        out_shape=[
            jax.ShapeDtypeStruct((n, d), jnp.float32),
            jax.ShapeDtypeStruct((n, d), jnp.bfloat16),
            jax.ShapeDtypeStruct((n, d), jnp.bfloat16),
        ],
    )(inputx, weight_mlp, weight_A, weight_As)

    outA, outAs, colsum = pl.pallas_call(
        _main_body,
        grid=(ni,),
        in_specs=[
            pl.BlockSpec((bm, n), lambda i: (i, 0)),
            pl.BlockSpec((bm, n), lambda i: (i, 0)),
            pl.BlockSpec((n, d), lambda i: (0, 0)),
            pl.BlockSpec((n, d), lambda i: (0, 0)),
            pl.BlockSpec((bm, d), lambda i: (i, 0)),
        ],
        out_specs=[
            pl.BlockSpec((bm, d), lambda i: (i, 0)),
            pl.BlockSpec((bm, d), lambda i: (i, 0)),
            pl.BlockSpec((8, d), lambda i: (0, 0)),
        ],
        out_shape=[
            jax.ShapeDtypeStruct((n, d), jnp.float32),
            jax.ShapeDtypeStruct((n, d), jnp.float32),
            jax.ShapeDtypeStruct((8, d), jnp.float32),
        ],
    )(adj, sadj, xwA, xwAs, mlp)

    attv_pad = jnp.zeros((8, 128), jnp.float32).at[:3, :3].set(att_vec_v)

    bm2 = n // 5 if n % 5 == 0 else n
    emb = pl.pallas_call(
        functools.partial(_attn_body, float(n)),
        grid=(n // bm2,),
        in_specs=[
            pl.BlockSpec((bm2, d), lambda i: (i, 0)),
            pl.BlockSpec((bm2, d), lambda i: (i, 0)),
            pl.BlockSpec((bm2, d), lambda i: (i, 0)),
            pl.BlockSpec((8, d), lambda i: (0, 0)),
            pl.BlockSpec((d, d), lambda i: (0, 0)),
            pl.BlockSpec((8, 128), lambda i: (0, 0)),
        ],
        out_specs=pl.BlockSpec((bm2, d), lambda i: (i, 0)),
        out_shape=jax.ShapeDtypeStruct((n, d), jnp.float32),
    )(mlp, outA, outAs, colsum, att_vec_k, attv_pad)

    return emb


# speedup vs baseline: 1.0338x; 1.0338x over previous
import jax
import jax.numpy as jnp
from jax.experimental import pallas as pl
from jax.experimental.pallas import tpu as pltpu


def _main_body(adj_ref, sadj_ref, x_full_ref, x_blk_ref, wmlp_ref,
               wA_ref, wAs_ref, outA_ref, outAs_ref, colsum_ref,
               xwA_s, xwAs_s):
    i = pl.program_id(0)

    @pl.when(i == 0)
    def _precompute():
        xf = x_full_ref[...]
        xwA_s[...] = jnp.dot(
            xf, wA_ref[...],
            preferred_element_type=jnp.float32).astype(jnp.bfloat16)
        xwAs_s[...] = jnp.dot(
            xf, wAs_ref[...],
            preferred_element_type=jnp.float32).astype(jnp.bfloat16)

    a = jnp.maximum(
        jnp.dot(adj_ref[...].astype(jnp.bfloat16), xwA_s[...],
                preferred_element_type=jnp.float32), 0.0)
    b = jnp.maximum(
        jnp.dot(sadj_ref[...].astype(jnp.bfloat16), xwAs_s[...],
                preferred_element_type=jnp.float32), 0.0)
    outA_ref[...] = a
    outAs_ref[...] = b
    mlp = jnp.maximum(
        jnp.dot(x_blk_ref[...], wmlp_ref[...],
                preferred_element_type=jnp.float32), 0.0)
    part = jnp.sum(a + b + mlp, axis=0, keepdims=True)

    @pl.when(i == 0)
    def _set():
        colsum_ref[0:1, :] = part

    @pl.when(i > 0)
    def _add():
        colsum_ref[0:1, :] += part


def _attn_body(n_total, outA_ref, outAs_ref, x_blk_ref, wmlp_ref,
               colsum_ref, attk_ref, attv_ref, emb_ref):
    tao = 3.0
    kvec = jnp.dot(colsum_ref[0:1, :] * (1.0 / n_total), attk_ref[...],
                   preferred_element_type=jnp.float32)
    mlp = jnp.maximum(
        jnp.dot(x_blk_ref[...], wmlp_ref[...],
                preferred_element_type=jnp.float32), 0.0)
    oA = outA_ref[...]
    oAs = outAs_ref[...]
    s0 = jnp.sum(mlp * kvec, axis=1, keepdims=True)
    s1 = jnp.sum(oA * kvec, axis=1, keepdims=True)
    s2 = jnp.sum(oAs * kvec, axis=1, keepdims=True)
    g0 = jax.nn.sigmoid(s0)
    g1 = jax.nn.sigmoid(s1)
    g2 = jax.nn.sigmoid(s2)
    v = attv_ref
    t0 = (g0 * v[0:1, 0:1] + g1 * v[1:2, 0:1] + g2 * v[2:3, 0:1]) * (1.0 / tao)
    t1 = (g0 * v[0:1, 1:2] + g1 * v[1:2, 1:2] + g2 * v[2:3, 1:2]) * (1.0 / tao)
    t2 = (g0 * v[0:1, 2:3] + g1 * v[1:2, 2:3] + g2 * v[2:3, 2:3]) * (1.0 / tao)
    m = jnp.maximum(t0, jnp.maximum(t1, t2))
    e0 = jnp.exp(t0 - m)
    e1 = jnp.exp(t1 - m)
    e2 = jnp.exp(t2 - m)
    den = e0 + e1 + e2
    emb_ref[...] = (e0 * mlp + e1 * oA + e2 * oAs) / den


def kernel(inputx, adj, sadj, weight_mlp, weight_A, weight_As,
           att_vec_k, att_vec_v):
    n, d = inputx.shape

    bm = 200 if n % 200 == 0 else n
    ni = n // bm

    outA, outAs, colsum = pl.pallas_call(
        _main_body,
        grid=(ni,),
        in_specs=[
            pl.BlockSpec((bm, n), lambda i: (i, 0)),
            pl.BlockSpec((bm, n), lambda i: (i, 0)),
            pl.BlockSpec((n, d), lambda i: (0, 0)),
            pl.BlockSpec((bm, d), lambda i: (i, 0)),
            pl.BlockSpec((d, d), lambda i: (0, 0)),
            pl.BlockSpec((d, d), lambda i: (0, 0)),
            pl.BlockSpec((d, d), lambda i: (0, 0)),
        ],
        out_specs=[
            pl.BlockSpec((bm, d), lambda i: (i, 0)),
            pl.BlockSpec((bm, d), lambda i: (i, 0)),
            pl.BlockSpec((8, d), lambda i: (0, 0)),
        ],
        out_shape=[
            jax.ShapeDtypeStruct((n, d), jnp.float32),
            jax.ShapeDtypeStruct((n, d), jnp.float32),
            jax.ShapeDtypeStruct((8, d), jnp.float32),
        ],
        scratch_shapes=[
            pltpu.VMEM((n, d), jnp.bfloat16),
            pltpu.VMEM((n, d), jnp.bfloat16),
        ],
        compiler_params=pltpu.CompilerParams(
            vmem_limit_bytes=63 * 1024 * 1024),
    )(adj, sadj, inputx, inputx, weight_mlp, weight_A, weight_As)

    attv_pad = jnp.zeros((8, 128), jnp.float32).at[:3, :3].set(att_vec_v)

    bm2 = n // 5 if n % 5 == 0 else n
    emb = pl.pallas_call(
        lambda *refs: _attn_body(float(n), *refs),
        grid=(n // bm2,),
        in_specs=[
            pl.BlockSpec((bm2, d), lambda i: (i, 0)),
            pl.BlockSpec((bm2, d), lambda i: (i, 0)),
            pl.BlockSpec((bm2, d), lambda i: (i, 0)),
            pl.BlockSpec((d, d), lambda i: (0, 0)),
            pl.BlockSpec((8, d), lambda i: (0, 0)),
            pl.BlockSpec((d, d), lambda i: (0, 0)),
            pl.BlockSpec((8, 128), lambda i: (0, 0)),
        ],
        out_specs=pl.BlockSpec((bm2, d), lambda i: (i, 0)),
        out_shape=jax.ShapeDtypeStruct((n, d), jnp.float32),
    )(outA, outAs, inputx, weight_mlp, colsum, att_vec_k, attv_pad)

    return emb


# speedup vs baseline: 1.0398x; 1.0057x over previous
import jax
import jax.numpy as jnp
from jax.experimental import pallas as pl
from jax.experimental.pallas import tpu as pltpu


def _main_body(adj_ref, sadj_ref, x_full_ref, x_blk_ref, wmlp_ref,
               wA_ref, wAs_ref, outA_ref, outAs_ref, colsum_ref,
               xwA_s, xwAs_s):
    i = pl.program_id(0)

    @pl.when(i == 0)
    def _precompute():
        xf = x_full_ref[...]
        xwA_s[...] = jnp.dot(
            xf, wA_ref[...],
            preferred_element_type=jnp.float32).astype(jnp.bfloat16)
        xwAs_s[...] = jnp.dot(
            xf, wAs_ref[...],
            preferred_element_type=jnp.float32).astype(jnp.bfloat16)

    a = jnp.maximum(
        jnp.dot(adj_ref[...].astype(jnp.bfloat16), xwA_s[...],
                preferred_element_type=jnp.float32), 0.0)
    b = jnp.maximum(
        jnp.dot(sadj_ref[...].astype(jnp.bfloat16), xwAs_s[...],
                preferred_element_type=jnp.float32), 0.0)
    outA_ref[...] = a.astype(jnp.bfloat16)
    outAs_ref[...] = b.astype(jnp.bfloat16)
    mlp = jnp.maximum(
        jnp.dot(x_blk_ref[...], wmlp_ref[...],
                preferred_element_type=jnp.float32), 0.0)
    part = jnp.sum(a + b + mlp, axis=0, keepdims=True)

    @pl.when(i == 0)
    def _set():
        colsum_ref[0:1, :] = part

    @pl.when(i > 0)
    def _add():
        colsum_ref[0:1, :] += part


def _attn_body(n_total, outA_ref, outAs_ref, x_blk_ref, wmlp_ref,
               colsum_ref, attk_ref, attv_ref, emb_ref):
    tao = 3.0
    kvec = jnp.dot(colsum_ref[0:1, :] * (1.0 / n_total), attk_ref[...],
                   preferred_element_type=jnp.float32)
    mlp = jnp.maximum(
        jnp.dot(x_blk_ref[...], wmlp_ref[...],
                preferred_element_type=jnp.float32), 0.0)
    oA = outA_ref[...].astype(jnp.float32)
    oAs = outAs_ref[...].astype(jnp.float32)
    s0 = jnp.sum(mlp * kvec, axis=1, keepdims=True)
    s1 = jnp.sum(oA * kvec, axis=1, keepdims=True)
    s2 = jnp.sum(oAs * kvec, axis=1, keepdims=True)
    g0 = jax.nn.sigmoid(s0)
    g1 = jax.nn.sigmoid(s1)
    g2 = jax.nn.sigmoid(s2)
    v = attv_ref
    t0 = (g0 * v[0:1, 0:1] + g1 * v[1:2, 0:1] + g2 * v[2:3, 0:1]) * (1.0 / tao)
    t1 = (g0 * v[0:1, 1:2] + g1 * v[1:2, 1:2] + g2 * v[2:3, 1:2]) * (1.0 / tao)
    t2 = (g0 * v[0:1, 2:3] + g1 * v[1:2, 2:3] + g2 * v[2:3, 2:3]) * (1.0 / tao)
    m = jnp.maximum(t0, jnp.maximum(t1, t2))
    e0 = jnp.exp(t0 - m)
    e1 = jnp.exp(t1 - m)
    e2 = jnp.exp(t2 - m)
    den = e0 + e1 + e2
    emb_ref[...] = (e0 * mlp + e1 * oA + e2 * oAs) / den


def kernel(inputx, adj, sadj, weight_mlp, weight_A, weight_As,
           att_vec_k, att_vec_v):
    n, d = inputx.shape

    bm = 200 if n % 200 == 0 else n
    ni = n // bm

    outA, outAs, colsum = pl.pallas_call(
        _main_body,
        grid=(ni,),
        in_specs=[
            pl.BlockSpec((bm, n), lambda i: (i, 0)),
            pl.BlockSpec((bm, n), lambda i: (i, 0)),
            pl.BlockSpec((n, d), lambda i: (0, 0)),
            pl.BlockSpec((bm, d), lambda i: (i, 0)),
            pl.BlockSpec((d, d), lambda i: (0, 0)),
            pl.BlockSpec((d, d), lambda i: (0, 0)),
            pl.BlockSpec((d, d), lambda i: (0, 0)),
        ],
        out_specs=[
            pl.BlockSpec((bm, d), lambda i: (i, 0)),
            pl.BlockSpec((bm, d), lambda i: (i, 0)),
            pl.BlockSpec((8, d), lambda i: (0, 0)),
        ],
        out_shape=[
            jax.ShapeDtypeStruct((n, d), jnp.bfloat16),
            jax.ShapeDtypeStruct((n, d), jnp.bfloat16),
            jax.ShapeDtypeStruct((8, d), jnp.float32),
        ],
        scratch_shapes=[
            pltpu.VMEM((n, d), jnp.bfloat16),
            pltpu.VMEM((n, d), jnp.bfloat16),
        ],
        compiler_params=pltpu.CompilerParams(
            vmem_limit_bytes=63 * 1024 * 1024),
    )(adj, sadj, inputx, inputx, weight_mlp, weight_A, weight_As)

    attv_pad = jnp.zeros((8, 128), jnp.float32).at[:3, :3].set(att_vec_v)

    bm2 = n // 5 if n % 5 == 0 else n
    emb = pl.pallas_call(
        lambda *refs: _attn_body(float(n), *refs),
        grid=(n // bm2,),
        in_specs=[
            pl.BlockSpec((bm2, d), lambda i: (i, 0)),
            pl.BlockSpec((bm2, d), lambda i: (i, 0)),
            pl.BlockSpec((bm2, d), lambda i: (i, 0)),
            pl.BlockSpec((d, d), lambda i: (0, 0)),
            pl.BlockSpec((8, d), lambda i: (0, 0)),
            pl.BlockSpec((d, d), lambda i: (0, 0)),
            pl.BlockSpec((8, 128), lambda i: (0, 0)),
        ],
        out_specs=pl.BlockSpec((bm2, d), lambda i: (i, 0)),
        out_shape=jax.ShapeDtypeStruct((n, d), jnp.float32),
    )(outA, outAs, inputx, weight_mlp, colsum, att_vec_k, attv_pad)

    return emb
